# baseline (speedup 1.0000x reference)
import jax
import jax.numpy as jnp
from jax.experimental import pallas as pl

_ROWS = 1048576
_COLS = 64
_BR = 8192


def _body(x_ref, o_ref):
    blk = x_ref[...]

    @pl.when(pl.program_id(0) == 0)
    def _():
        row = jax.lax.broadcasted_iota(jnp.int32, (_BR, _COLS), 0)
        o_ref[...] = jnp.where(row == 2, jnp.float32(9.0), blk)

    @pl.when(pl.program_id(0) != 0)
    def _():
        o_ref[...] = blk


def kernel(x):
    return pl.pallas_call(
        _body,
        grid=(_ROWS // _BR,),
        in_specs=[pl.BlockSpec((_BR, _COLS), lambda i: (i, 0))],
        out_specs=pl.BlockSpec((_BR, _COLS), lambda i: (i, 0)),
        out_shape=jax.ShapeDtypeStruct((_ROWS, _COLS), jnp.float32),
    )(x)
